# Initial kernel scaffold; baseline (speedup 1.0000x reference)
import functools

import jax
import jax.numpy as jnp
from jax import lax
from jax.experimental import pallas as pl
from jax.experimental.pallas import tpu as pltpu
from jax.experimental.pallas import tpu_sc as plsc

N = 10000
NP = 10240
E = 320000
D = 128
ALPHA = 0.2

NW = 32
C = E // NW
CH = 80
NCH = C // CH
ER = E // CH
SEG = NP // 16
F32 = jnp.float32
I32 = jnp.int32

_MESH = plsc.VectorSubcoreMesh(core_axis_name="c", subcore_axis_name="s")


def _node_body(x_ref, w_ref, a1_ref, a2_ref, h_ref, u1_ref, w2_ref):
    h = jnp.dot(x_ref[...], w_ref[...], preferred_element_type=F32)
    h_ref[...] = h
    u1_ref[...] = jnp.sum(h * a1_ref[0, :D][None, :], axis=1)
    w2_ref[...] = jnp.sum(h * a2_ref[0, D:][None, :], axis=1)


def _node_pre(x_pad, W, a1, a2):
    BM = 2048
    return pl.pallas_call(
        _node_body,
        grid=(NP // BM,),
        in_specs=[pl.BlockSpec((BM, D), lambda i: (i, 0)),
                  pl.BlockSpec((D, D), lambda i: (0, 0)),
                  pl.BlockSpec((1, 2 * D), lambda i: (0, 0)),
                  pl.BlockSpec((1, 2 * D), lambda i: (0, 0))],
        out_specs=[pl.BlockSpec((BM, D), lambda i: (i, 0)),
                   pl.BlockSpec((BM,), lambda i: (i,)),
                   pl.BlockSpec((BM,), lambda i: (i,))],
        out_shape=[jax.ShapeDtypeStruct((NP, D), F32),
                   jax.ShapeDtypeStruct((NP,), F32),
                   jax.ShapeDtypeStruct((NP,), F32)],
    )(x_pad, W, a1, a2)


def _edge_body(p_ref, nh_ref, w_ref, a1_ref, a2_ref, v1_ref, u2_ref):
    W = w_ref[...]
    c1 = jnp.sum(W * a1_ref[0, D:][None, :], axis=1)
    c2 = jnp.sum(W * a2_ref[0, :D][None, :], axis=1)
    v1_ref[...] = jnp.sum(p_ref[...] * c1[None, :], axis=1)
    u2_ref[...] = jnp.sum(nh_ref[...] * c2[None, :], axis=1)


def _edge_pre(p_h, new_h, W, a1, a2):
    BM = 6400
    return pl.pallas_call(
        _edge_body,
        grid=(E // BM,),
        in_specs=[pl.BlockSpec((BM, D), lambda i: (i, 0)),
                  pl.BlockSpec((BM, D), lambda i: (i, 0)),
                  pl.BlockSpec((D, D), lambda i: (0, 0)),
                  pl.BlockSpec((1, 2 * D), lambda i: (0, 0)),
                  pl.BlockSpec((1, 2 * D), lambda i: (0, 0))],
        out_specs=[pl.BlockSpec((BM,), lambda i: (i,)),
                   pl.BlockSpec((BM,), lambda i: (i,))],
        out_shape=[jax.ShapeDtypeStruct((E,), F32),
                   jax.ShapeDtypeStruct((E,), F32)],
    )(p_h, new_h, W, a1, a2)


def _combine_body(p0_ref, p1_ref, o_ref):
    hp = p0_ref[...] + p1_ref[...]
    o_ref[...] = jnp.where(hp > 0, hp, jnp.expm1(hp))


def _combine(p0, p1):
    BM = 2000
    return pl.pallas_call(
        _combine_body,
        grid=(N // BM,),
        in_specs=[pl.BlockSpec((BM, D), lambda i: (i, 0)),
                  pl.BlockSpec((BM, D), lambda i: (i, 0))],
        out_specs=pl.BlockSpec((BM, D), lambda i: (i, 0)),
        out_shape=jax.ShapeDtypeStruct((N, D), F32),
    )(p0, p1)


def _worker():
    cid = lax.axis_index("c")
    sid = lax.axis_index("s")
    return cid, sid, cid * 16 + sid


def _leaky_exp(s):
    return jnp.exp(-jnp.where(s >= 0, s, ALPHA * s))


def _sc_attn_col(u1, v1_2d, ec0_2d, row_2d):
    @functools.partial(
        pl.kernel,
        out_type=[jax.ShapeDtypeStruct((ER, CH), F32),
                  jax.ShapeDtypeStruct((2, NP), F32)],
        mesh=_MESH,
        scratch_types=[
            pltpu.VMEM((NP,), F32),
            pltpu.VMEM((NCH, CH), F32),
            pltpu.VMEM((NCH, CH), I32),
            pltpu.VMEM((NCH, CH), I32),
            pltpu.VMEM((NCH, CH), F32),
            pltpu.VMEM((SEG,), F32),
            pltpu.VMEM_SHARED((NP,), F32),
        ],
    )
    def kern(u1_hbm, v1_hbm, ec0_hbm, row_hbm, ex1_hbm, part_hbm,
             u1_v, v1_v, ec0_v, row_v, ex1_v, zb_v, seg_sh):
        cid, sid, wid = _worker()
        rbase = wid * NCH
        pltpu.sync_copy(u1_hbm, u1_v)
        pltpu.sync_copy(v1_hbm.at[pl.ds(rbase, NCH)], v1_v)
        pltpu.sync_copy(ec0_hbm.at[pl.ds(rbase, NCH)], ec0_v)
        pltpu.sync_copy(row_hbm.at[pl.ds(rbase, NCH)], row_v)

        def zfill(i, _):
            zb_v[pl.ds(i * 16, 16)] = jnp.zeros((16,), F32)
            return 0
        lax.fori_loop(0, SEG // 16, zfill, 0)
        pltpu.sync_copy(zb_v, seg_sh.at[pl.ds(sid * SEG, SEG)])
        plsc.subcore_barrier()

        def row_fn(r, _):
            for k in range(CH // 16):
                sl = pl.ds(k * 16, 16)
                g = plsc.load_gather(u1_v, [ec0_v[r, sl]])
                ec = _leaky_exp(g + v1_v[r, sl])
                ex1_v[r, sl] = jnp.exp(ec)
            return 0
        lax.fori_loop(0, NCH, row_fn, 0)
        pltpu.sync_copy(ex1_v, ex1_hbm.at[pl.ds(rbase, NCH)])
        pltpu.sync_copy(ex1_v, seg_sh.at[row_v], add=True)
        plsc.subcore_barrier()
        pltpu.sync_copy(seg_sh.at[pl.ds(sid * SEG, SEG)],
                        part_hbm.at[cid, pl.ds(sid * SEG, SEG)])

    return kern(u1, v1_2d, ec0_2d, row_2d)


def _sc_softmax_div(seg_part, ex1_2d, row_2d):
    @functools.partial(
        pl.kernel,
        out_type=jax.ShapeDtypeStruct((ER, CH), F32),
        mesh=_MESH,
        scratch_types=[
            pltpu.VMEM((NP,), F32),
            pltpu.VMEM((NP,), F32),
            pltpu.VMEM((NCH, CH), F32),
            pltpu.VMEM((NCH, CH), I32),
            pltpu.VMEM((NCH, CH), F32),
        ],
    )
    def kern(part_hbm, ex1_hbm, row_hbm, z_hbm,
             pa_v, pb_v, ex1_v, row_v, z_v):
        cid, sid, wid = _worker()
        rbase = wid * NCH
        pltpu.sync_copy(part_hbm.at[0], pa_v)
        pltpu.sync_copy(part_hbm.at[1], pb_v)
        pltpu.sync_copy(ex1_hbm.at[pl.ds(rbase, NCH)], ex1_v)
        pltpu.sync_copy(row_hbm.at[pl.ds(rbase, NCH)], row_v)

        def red(i, _):
            sl = pl.ds(i * 16, 16)
            pa_v[sl] = pa_v[sl] + pb_v[sl] + 1e-16
            return 0
        lax.fori_loop(0, NP // 16, red, 0)

        def row_fn(r, _):
            for k in range(CH // 16):
                sl = pl.ds(k * 16, 16)
                ss = plsc.load_gather(pa_v, [row_v[r, sl]])
                z_v[r, sl] = ex1_v[r, sl] / ss
            return 0
        lax.fori_loop(0, NCH, row_fn, 0)
        pltpu.sync_copy(z_v, z_hbm.at[pl.ds(rbase, NCH)])

    return kern(seg_part, ex1_2d, row_2d)


def _sc_edge_row(w2n, u2_flat, z_flat, rr_2d, e1_2d, e0_2d):
    @functools.partial(
        pl.kernel,
        out_type=[jax.ShapeDtypeStruct((ER, CH), F32),
                  jax.ShapeDtypeStruct((2, NP), F32)],
        mesh=_MESH,
        scratch_types=[
            pltpu.VMEM((NP,), F32),
            pltpu.VMEM((NCH, CH), I32),
            pltpu.VMEM((NCH, CH), I32),
            pltpu.VMEM((NCH, CH), I32),
            pltpu.VMEM((NCH, CH), F32),
            pltpu.VMEM((NCH, CH), F32),
            pltpu.VMEM((NCH, CH), F32),
            pltpu.VMEM((SEG,), F32),
            pltpu.VMEM_SHARED((NP,), F32),
            pltpu.SemaphoreType.DMA,
        ],
    )
    def kern(w2_hbm, u2_hbm, z_hbm, rr_hbm, e1_hbm, e0_hbm, ee_hbm, part_hbm,
             w2_v, rr_v, e1_v, e0_v, u2r_v, zr_v, ee_v, zb_v, rs_sh, sem):
        cid, sid, wid = _worker()
        rbase = wid * NCH
        pltpu.sync_copy(w2_hbm, w2_v)
        pltpu.sync_copy(rr_hbm.at[pl.ds(rbase, NCH)], rr_v)
        pltpu.sync_copy(e1_hbm.at[pl.ds(rbase, NCH)], e1_v)
        pltpu.sync_copy(e0_hbm.at[pl.ds(rbase, NCH)], e0_v)

        def zfill(i, _):
            zb_v[pl.ds(i * 16, 16)] = jnp.zeros((16,), F32)
            return 0
        lax.fori_loop(0, SEG // 16, zfill, 0)
        pltpu.sync_copy(zb_v, rs_sh.at[pl.ds(sid * SEG, SEG)])
        plsc.subcore_barrier()

        cp1 = pltpu.async_copy(u2_hbm.at[rr_v], u2r_v, sem)
        cp2 = pltpu.async_copy(z_hbm.at[rr_v], zr_v, sem)
        cp1.wait()
        cp2.wait()

        def row_fn(r, _):
            for k in range(CH // 16):
                sl = pl.ds(k * 16, 16)
                g = plsc.load_gather(w2_v, [e1_v[r, sl]])
                er = _leaky_exp(u2r_v[r, sl] + g)
                ee_v[r, sl] = er * zr_v[r, sl]
            return 0
        lax.fori_loop(0, NCH, row_fn, 0)
        pltpu.sync_copy(ee_v, ee_hbm.at[pl.ds(rbase, NCH)])
        pltpu.sync_copy(ee_v, rs_sh.at[e0_v], add=True)
        plsc.subcore_barrier()
        pltpu.sync_copy(rs_sh.at[pl.ds(sid * SEG, SEG)],
                        part_hbm.at[cid, pl.ds(sid * SEG, SEG)])

    return kern(w2n, u2_flat, z_flat, rr_2d, e1_2d, e0_2d)


def _sc_aggregate(h_pad, ee_2d, rs_part, e0_2d, e1_2d):
    @functools.partial(
        pl.kernel,
        out_type=[jax.ShapeDtypeStruct((ER, CH), F32),
                  jax.ShapeDtypeStruct((2, NP, D), F32)],
        mesh=_MESH,
        scratch_types=[
            pltpu.VMEM((NP,), F32),
            pltpu.VMEM((NP,), F32),
            pltpu.VMEM((NCH, CH), I32),
            pltpu.VMEM((NCH, CH), I32),
            pltpu.VMEM((NCH, CH), F32),
            pltpu.VMEM((NCH, CH), F32),
            pltpu.VMEM((CH, D), F32),
            pltpu.VMEM((16, D), F32),
            pltpu.VMEM_SHARED((NP, D), F32),
            pltpu.SemaphoreType.DMA,
        ],
    )
    def kern(h_hbm, ee_hbm, part_hbm, e0_hbm, e1_hbm, att_hbm, hp_hbm,
             p0_v, p1_v, e0_v, e1_v, ee_v, att_v, rows_v, zb_v, hp_sh, sem):
        cid, sid, wid = _worker()
        rbase = wid * NCH
        pltpu.sync_copy(part_hbm.at[0], p0_v)
        pltpu.sync_copy(part_hbm.at[1], p1_v)
        pltpu.sync_copy(ee_hbm.at[pl.ds(rbase, NCH)], ee_v)
        pltpu.sync_copy(e0_hbm.at[pl.ds(rbase, NCH)], e0_v)
        pltpu.sync_copy(e1_hbm.at[pl.ds(rbase, NCH)], e1_v)

        def red(i, _):
            sl = pl.ds(i * 16, 16)
            s = p0_v[sl] + p1_v[sl]
            p0_v[sl] = jnp.where(s == 0, 1.0, s)
            return 0
        lax.fori_loop(0, NP // 16, red, 0)

        for i in range(16):
            for k in range(D // 16):
                zb_v[i, pl.ds(k * 16, 16)] = jnp.zeros((16,), F32)

        def zrow(j, _):
            pltpu.sync_copy(zb_v, hp_sh.at[pl.ds(sid * SEG + j * 16, 16)])
            return 0
        lax.fori_loop(0, SEG // 16, zrow, 0)
        plsc.subcore_barrier()

        def row_fn(r, _):
            for k in range(CH // 16):
                sl = pl.ds(k * 16, 16)
                rs = plsc.load_gather(p0_v, [e0_v[r, sl]])
                att_v[r, sl] = ee_v[r, sl] / rs
            return 0
        lax.fori_loop(0, NCH, row_fn, 0)
        pltpu.sync_copy(att_v, att_hbm.at[pl.ds(rbase, NCH)])

        def chunk_fn(ch, _):
            pltpu.async_copy(h_hbm.at[e1_v.at[ch]], rows_v, sem).wait()

            def srow(r, _):
                a = plsc.load_gather(
                    att_v, [jnp.full((16,), ch, I32), jnp.full((16,), r, I32)])
                for k in range(D // 16):
                    sl = pl.ds(k * 16, 16)
                    rows_v[r, sl] = rows_v[r, sl] * a
                return 0
            lax.fori_loop(0, CH, srow, 0)
            pltpu.sync_copy(rows_v, hp_sh.at[e0_v.at[ch]], add=True)
            return 0
        lax.fori_loop(0, NCH, chunk_fn, 0)
        plsc.subcore_barrier()
        pltpu.sync_copy(hp_sh.at[pl.ds(sid * SEG, SEG)],
                        hp_hbm.at[cid, pl.ds(sid * SEG, SEG)])

    return kern(h_pad, ee_2d, rs_part, e0_2d, e1_2d)


def kernel(input, adj, edge, p_h, edge_col, row_i, row_resort, new_h, W, a1, a2):
    x_pad = jnp.pad(input, ((0, NP - N), (0, 0)))
    h_pad, u1, w2n = _node_pre(x_pad, W, a1, a2)
    v1, u2 = _edge_pre(p_h, new_h, W, a1, a2)

    ec0_2d = edge_col[0].reshape(ER, CH)
    row_2d = row_i.reshape(ER, CH)
    rr_2d = row_resort.reshape(ER, CH)
    e0_2d = edge[0].reshape(ER, CH)
    e1_2d = edge[1].reshape(ER, CH)

    ex1_2d, seg_part = _sc_attn_col(u1, v1.reshape(ER, CH), ec0_2d, row_2d)
    z_2d = _sc_softmax_div(seg_part, ex1_2d, row_2d)
    ee_2d, rs_part = _sc_edge_row(w2n, u2, z_2d.reshape(E), rr_2d, e1_2d, e0_2d)
    att_2d, hp_part = _sc_aggregate(h_pad, ee_2d, rs_part, e0_2d, e1_2d)

    h_prime = _combine(hp_part[0, :N], hp_part[1, :N])
    return h_prime, edge, att_2d.reshape(E, 1)


# speedup vs baseline: 6.2673x; 6.2673x over previous
import functools

import jax
import jax.numpy as jnp
from jax import lax
from jax.experimental import pallas as pl
from jax.experimental.pallas import tpu as pltpu
from jax.experimental.pallas import tpu_sc as plsc

N = 10000
NP = 10240
E = 320000
D = 128
ALPHA = 0.2

NW = 32
C = E // NW
CH = 80
NCH = C // CH
ER = E // CH
SEG = NP // 16
F32 = jnp.float32
I32 = jnp.int32

_mesh_cache = []


def _MESH():
    if not _mesh_cache:
        _mesh_cache.append(plsc.VectorSubcoreMesh(
            core_axis_name="c", subcore_axis_name="s"))
    return _mesh_cache[0]


def _node_body(x_ref, w_ref, a1_ref, a2_ref, h_ref, u1_ref, w2_ref):
    h = jnp.dot(x_ref[...], w_ref[...], preferred_element_type=F32)
    h_ref[...] = h
    u1_ref[...] = jnp.sum(h * a1_ref[0, :D][None, :], axis=1)
    w2_ref[...] = jnp.sum(h * a2_ref[0, D:][None, :], axis=1)


def _node_pre(x_pad, W, a1, a2):
    BM = 2048
    return pl.pallas_call(
        _node_body,
        grid=(NP // BM,),
        in_specs=[pl.BlockSpec((BM, D), lambda i: (i, 0)),
                  pl.BlockSpec((D, D), lambda i: (0, 0)),
                  pl.BlockSpec((1, 2 * D), lambda i: (0, 0)),
                  pl.BlockSpec((1, 2 * D), lambda i: (0, 0))],
        out_specs=[pl.BlockSpec((BM, D), lambda i: (i, 0)),
                   pl.BlockSpec((BM,), lambda i: (i,)),
                   pl.BlockSpec((BM,), lambda i: (i,))],
        out_shape=[jax.ShapeDtypeStruct((NP, D), F32),
                   jax.ShapeDtypeStruct((NP,), F32),
                   jax.ShapeDtypeStruct((NP,), F32)],
    )(x_pad, W, a1, a2)


def _edge_body(p_ref, nh_ref, w_ref, a1_ref, a2_ref, v1_ref, u2_ref):
    W = w_ref[...]
    c1 = jnp.sum(W * a1_ref[0, D:][None, :], axis=1)
    c2 = jnp.sum(W * a2_ref[0, :D][None, :], axis=1)
    v1_ref[...] = jnp.sum(p_ref[...] * c1[None, :], axis=1)
    u2_ref[...] = jnp.sum(nh_ref[...] * c2[None, :], axis=1)


def _edge_pre(p_h, new_h, W, a1, a2):
    BM = 512
    return pl.pallas_call(
        _edge_body,
        grid=(E // BM,),
        in_specs=[pl.BlockSpec((BM, D), lambda i: (i, 0)),
                  pl.BlockSpec((BM, D), lambda i: (i, 0)),
                  pl.BlockSpec((D, D), lambda i: (0, 0)),
                  pl.BlockSpec((1, 2 * D), lambda i: (0, 0)),
                  pl.BlockSpec((1, 2 * D), lambda i: (0, 0))],
        out_specs=[pl.BlockSpec((BM,), lambda i: (i,)),
                   pl.BlockSpec((BM,), lambda i: (i,))],
        out_shape=[jax.ShapeDtypeStruct((E,), F32),
                   jax.ShapeDtypeStruct((E,), F32)],
    )(p_h, new_h, W, a1, a2)


def _combine_body(p_ref, o_ref):
    hp = p_ref[...]
    o_ref[...] = jnp.where(hp > 0, hp, jnp.exp(jnp.minimum(hp, 0.0)) - 1.0)


def _combine(p):
    BM = 2000
    return pl.pallas_call(
        _combine_body,
        grid=(N // BM,),
        in_specs=[pl.BlockSpec((BM, D), lambda i: (i, 0))],
        out_specs=pl.BlockSpec((BM, D), lambda i: (i, 0)),
        out_shape=jax.ShapeDtypeStruct((N, D), F32),
    )(p)


def _worker():
    cid = lax.axis_index("c")
    sid = lax.axis_index("s")
    return cid, sid, cid * 16 + sid


def _leaky_exp(s):
    return jnp.exp(-jnp.where(s >= 0, s, ALPHA * s))


def _sc_attn_col(u1, v1_2d, ec0_2d, row_2d):
    @functools.partial(
        pl.kernel,
        out_type=[jax.ShapeDtypeStruct((NW, NCH, CH), F32),
                  jax.ShapeDtypeStruct((2 * NP,), F32)],
        mesh=_MESH(),
        compiler_params=pltpu.CompilerParams(needs_layout_passes=False),
        scratch_types=[
            pltpu.VMEM((NP,), F32),
            pltpu.VMEM((NCH, CH), F32),
            pltpu.VMEM((NCH, CH), I32),
            pltpu.VMEM((NCH, CH), I32),
            pltpu.VMEM((NCH, CH), F32),
            pltpu.VMEM((SEG,), F32),
            pltpu.VMEM_SHARED((NP,), F32),
        ],
    )
    def kern(u1_hbm, v1_hbm, ec0_hbm, row_hbm, ex1_hbm, part_hbm,
             u1_v, v1_v, ec0_v, row_v, ex1_v, zb_v, seg_sh):
        cid, sid, wid = _worker()
        pltpu.sync_copy(u1_hbm, u1_v)
        pltpu.sync_copy(v1_hbm.at[wid], v1_v)
        pltpu.sync_copy(ec0_hbm.at[wid], ec0_v)
        pltpu.sync_copy(row_hbm.at[wid], row_v)

        def zfill(i, _):
            zb_v[pl.ds(i * 16, 16)] = jnp.zeros((16,), F32)
            return 0
        lax.fori_loop(0, SEG // 16, zfill, 0)
        pltpu.sync_copy(zb_v, seg_sh.at[pl.ds(sid * SEG, SEG)])
        plsc.subcore_barrier()

        def row_fn(r, _):
            for k in range(CH // 16):
                sl = pl.ds(k * 16, 16)
                g = plsc.load_gather(u1_v, [ec0_v[r, sl]])
                ec = _leaky_exp(g + v1_v[r, sl])
                ex1_v[r, sl] = jnp.exp(ec)
            return 0
        lax.fori_loop(0, NCH, row_fn, 0)
        pltpu.sync_copy(ex1_v, ex1_hbm.at[wid])

        def scat(ch, _):
            pltpu.sync_copy(ex1_v.at[ch], seg_sh.at[row_v.at[ch]], add=True)
            return 0
        lax.fori_loop(0, NCH, scat, 0)
        plsc.subcore_barrier()
        pltpu.sync_copy(seg_sh.at[pl.ds(sid * SEG, SEG)],
                        part_hbm.at[pl.ds(cid * NP + sid * SEG, SEG)])

    return kern(u1, v1_2d, ec0_2d, row_2d)


def _sc_softmax_div(seg_part, ex1_2d, row_2d):
    @functools.partial(
        pl.kernel,
        out_type=jax.ShapeDtypeStruct((NW, NCH, CH), F32),
        mesh=_MESH(),
        compiler_params=pltpu.CompilerParams(needs_layout_passes=False),
        scratch_types=[
            pltpu.VMEM((NP,), F32),
            pltpu.VMEM((NP,), F32),
            pltpu.VMEM((NCH, CH), F32),
            pltpu.VMEM((NCH, CH), I32),
            pltpu.VMEM((NCH, CH), F32),
        ],
    )
    def kern(part_hbm, ex1_hbm, row_hbm, z_hbm,
             pa_v, pb_v, ex1_v, row_v, z_v):
        cid, sid, wid = _worker()
        pltpu.sync_copy(part_hbm.at[pl.ds(0, NP)], pa_v)
        pltpu.sync_copy(part_hbm.at[pl.ds(NP, NP)], pb_v)
        pltpu.sync_copy(ex1_hbm.at[wid], ex1_v)
        pltpu.sync_copy(row_hbm.at[wid], row_v)

        def red(i, _):
            sl = pl.ds(i * 16, 16)
            pa_v[sl] = pa_v[sl] + pb_v[sl] + 1e-16
            return 0
        lax.fori_loop(0, NP // 16, red, 0)

        def row_fn(r, _):
            for k in range(CH // 16):
                sl = pl.ds(k * 16, 16)
                ss = plsc.load_gather(pa_v, [row_v[r, sl]])
                z_v[r, sl] = ex1_v[r, sl] / ss
            return 0
        lax.fori_loop(0, NCH, row_fn, 0)
        pltpu.sync_copy(z_v, z_hbm.at[wid])

    return kern(seg_part, ex1_2d, row_2d)


def _sc_edge_row(w2n, u2_flat, z_flat, rr_2d, e1_2d, e0_2d):
    @functools.partial(
        pl.kernel,
        out_type=[jax.ShapeDtypeStruct((NW, NCH, CH), F32),
                  jax.ShapeDtypeStruct((2 * NP,), F32)],
        mesh=_MESH(),
        compiler_params=pltpu.CompilerParams(needs_layout_passes=False),
        scratch_types=[
            pltpu.VMEM((NP,), F32),
            pltpu.VMEM((NCH, CH), I32),
            pltpu.VMEM((NCH, CH), I32),
            pltpu.VMEM((NCH, CH), I32),
            pltpu.VMEM((NCH, CH), F32),
            pltpu.VMEM((NCH, CH), F32),
            pltpu.VMEM((NCH, CH), F32),
            pltpu.VMEM((SEG,), F32),
            pltpu.VMEM_SHARED((NP,), F32),
            pltpu.SemaphoreType.DMA,
        ],
    )
    def kern(w2_hbm, u2_hbm, z_hbm, rr_hbm, e1_hbm, e0_hbm, ee_hbm, part_hbm,
             w2_v, rr_v, e1_v, e0_v, u2r_v, zr_v, ee_v, zb_v, rs_sh, sem):
        cid, sid, wid = _worker()
        pltpu.sync_copy(w2_hbm, w2_v)
        pltpu.sync_copy(rr_hbm.at[wid], rr_v)
        pltpu.sync_copy(e1_hbm.at[wid], e1_v)
        pltpu.sync_copy(e0_hbm.at[wid], e0_v)

        def zfill(i, _):
            zb_v[pl.ds(i * 16, 16)] = jnp.zeros((16,), F32)
            return 0
        lax.fori_loop(0, SEG // 16, zfill, 0)
        pltpu.sync_copy(zb_v, rs_sh.at[pl.ds(sid * SEG, SEG)])
        plsc.subcore_barrier()

        def gat(ch, _):
            cp1 = pltpu.async_copy(u2_hbm.at[rr_v.at[ch]], u2r_v.at[ch], sem)
            cp2 = pltpu.async_copy(z_hbm.at[rr_v.at[ch]], zr_v.at[ch], sem)
            cp1.wait()
            cp2.wait()
            return 0
        lax.fori_loop(0, NCH, gat, 0)

        def row_fn(r, _):
            for k in range(CH // 16):
                sl = pl.ds(k * 16, 16)
                g = plsc.load_gather(w2_v, [e1_v[r, sl]])
                er = _leaky_exp(u2r_v[r, sl] + g)
                ee_v[r, sl] = er * zr_v[r, sl]
            return 0
        lax.fori_loop(0, NCH, row_fn, 0)
        pltpu.sync_copy(ee_v, ee_hbm.at[wid])

        def scat(ch, _):
            pltpu.sync_copy(ee_v.at[ch], rs_sh.at[e0_v.at[ch]], add=True)
            return 0
        lax.fori_loop(0, NCH, scat, 0)
        plsc.subcore_barrier()
        pltpu.sync_copy(rs_sh.at[pl.ds(sid * SEG, SEG)],
                        part_hbm.at[pl.ds(cid * NP + sid * SEG, SEG)])

    return kern(w2n, u2_flat, z_flat, rr_2d, e1_2d, e0_2d)


def _sc_aggregate(h_pad, ee_g, rs_part, e0_g, e1_g):
    HALF = NP // 2
    GS = 25
    NG = (E // 16) // (GS * CH)
    SEGH = HALF // 16

    @functools.partial(
        pl.kernel,
        out_type=[jax.ShapeDtypeStruct((16 * NG, GS, CH), F32),
                  jax.ShapeDtypeStruct((2, HALF, D), F32)],
        mesh=_MESH(),
        compiler_params=pltpu.CompilerParams(needs_layout_passes=False),
        scratch_types=[
            pltpu.VMEM((NP,), F32),
            pltpu.VMEM((SEG,), F32),
            pltpu.VMEM((GS, CH), I32),
            pltpu.VMEM((GS, CH), I32),
            pltpu.VMEM((GS, CH), F32),
            pltpu.VMEM((CH, D), F32),
            pltpu.VMEM_SHARED((HALF, D), F32),
            pltpu.SemaphoreType.DMA,
        ],
    )
    def kern(h_hbm, ee_hbm, part_hbm, e0_hbm, e1_hbm, att_hbm, hp_hbm,
             rs_v, tmp_v, e0_v, e1_v, att_v, rows_v, hp_sh, sem):
        cid, sid, wid = _worker()
        pltpu.sync_copy(part_hbm.at[pl.ds(0, NP)], rs_v)

        def red(k, _):
            pltpu.sync_copy(part_hbm.at[pl.ds(NP + k * SEG, SEG)], tmp_v)

            def red2(j, _):
                sl = pl.ds(j * 16, 16)
                gl = pl.ds(k * SEG + j * 16, 16)
                t = rs_v[gl] + tmp_v[sl]
                rs_v[gl] = jnp.where(t == 0, 1.0, t)
                return 0
            lax.fori_loop(0, SEG // 16, red2, 0)
            return 0
        lax.fori_loop(0, 16, red, 0)

        for i in range(16):
            for k in range(D // 16):
                rows_v[i, pl.ds(k * 16, 16)] = jnp.zeros((16,), F32)

        def zrow(j, _):
            pltpu.sync_copy(rows_v.at[pl.ds(0, 16)],
                            hp_sh.at[pl.ds(sid * SEGH + j * 16, 16)])
            return 0
        lax.fori_loop(0, SEGH // 16, zrow, 0)
        plsc.subcore_barrier()

        base = cid * HALF

        def group_fn(g, _):
            gid = sid * NG + g
            pltpu.sync_copy(ee_hbm.at[gid], att_v)
            pltpu.sync_copy(e0_hbm.at[gid], e0_v)
            pltpu.sync_copy(e1_hbm.at[gid], e1_v)

            def att_fn(r, _):
                for k in range(CH // 16):
                    sl = pl.ds(k * 16, 16)
                    rs = plsc.load_gather(rs_v, [e0_v[r, sl]])
                    att_v[r, sl] = att_v[r, sl] / rs
                return 0
            lax.fori_loop(0, GS, att_fn, 0)

            @pl.when(cid == 0)
            def _():
                pltpu.sync_copy(att_v, att_hbm.at[gid])

            def mask_fn(r, _):
                for k in range(CH // 16):
                    sl = pl.ds(k * 16, 16)
                    idx = e0_v[r, sl] - base
                    inb = (idx >= 0) & (idx < HALF)
                    e0_v[r, sl] = jnp.where(inb, idx, 0)
                    att_v[r, sl] = jnp.where(inb, att_v[r, sl], 0.0)
                return 0
            lax.fori_loop(0, GS, mask_fn, 0)

            def chunk_fn(ch, _):
                pltpu.async_copy(h_hbm.at[e1_v.at[ch]], rows_v, sem).wait()

                def srow(r, _):
                    a = plsc.load_gather(
                        att_v,
                        [jnp.full((16,), ch, I32), jnp.full((16,), r, I32)])
                    for k in range(D // 16):
                        sl = pl.ds(k * 16, 16)
                        rows_v[r, sl] = rows_v[r, sl] * a
                    return 0
                lax.fori_loop(0, CH, srow, 0)
                pltpu.sync_copy(rows_v, hp_sh.at[e0_v.at[ch]], add=True)
                return 0
            lax.fori_loop(0, GS, chunk_fn, 0)
            return 0
        lax.fori_loop(0, NG, group_fn, 0)
        plsc.subcore_barrier()
        pltpu.sync_copy(hp_sh.at[pl.ds(sid * SEGH, SEGH)],
                        hp_hbm.at[cid, pl.ds(sid * SEGH, SEGH)])

    return kern(h_pad, ee_g, rs_part, e0_g, e1_g)


def kernel(input, adj, edge, p_h, edge_col, row_i, row_resort, new_h, W, a1, a2):
    x_pad = jnp.pad(input, ((0, NP - N), (0, 0)))
    h_pad, u1, w2n = _node_pre(x_pad, W, a1, a2)
    v1, u2 = _edge_pre(p_h, new_h, W, a1, a2)

    ec0_2d = edge_col[0].reshape(NW, NCH, CH)
    row_2d = row_i.reshape(NW, NCH, CH)
    rr_2d = row_resort.reshape(NW, NCH, CH)
    e0_2d = edge[0].reshape(NW, NCH, CH)
    e1_2d = edge[1].reshape(NW, NCH, CH)

    ex1_2d, seg_part = _sc_attn_col(u1, v1.reshape(NW, NCH, CH), ec0_2d, row_2d)
    z_2d = _sc_softmax_div(seg_part, ex1_2d, row_2d)
    ee_2d, rs_part = _sc_edge_row(w2n, u2, z_2d.reshape(E), rr_2d, e1_2d, e0_2d)
    gshape = (160, 25, CH)
    att_h, hp_halves = _sc_aggregate(h_pad, ee_2d.reshape(gshape), rs_part,
                                     edge[0].reshape(gshape),
                                     edge[1].reshape(gshape))

    h_prime = _combine(hp_halves.reshape(NP, D)[:N])
    return h_prime, edge, att_h.reshape(E, 1)


# speedup vs baseline: 10.0341x; 1.6010x over previous
import functools

import jax
import jax.numpy as jnp
from jax import lax
from jax.experimental import pallas as pl
from jax.experimental.pallas import tpu as pltpu
from jax.experimental.pallas import tpu_sc as plsc

N = 10000
NP = 10240
E = 320000
D = 128
ALPHA = 0.2

NW = 32
C = E // NW
CH = 80
NCH = C // CH
ER = E // CH
SEG = NP // 16
F32 = jnp.float32
I32 = jnp.int32

_mesh_cache = []


def _MESH():
    if not _mesh_cache:
        _mesh_cache.append(plsc.VectorSubcoreMesh(
            core_axis_name="c", subcore_axis_name="s"))
    return _mesh_cache[0]


def _node_body(x_ref, w_ref, a1_ref, a2_ref, h_ref, u1_ref, w2_ref):
    h = jnp.dot(x_ref[...], w_ref[...], preferred_element_type=F32)
    h_ref[...] = h
    u1_ref[...] = jnp.sum(h * a1_ref[0, :D][None, :], axis=1)
    w2_ref[...] = jnp.sum(h * a2_ref[0, D:][None, :], axis=1)


def _node_pre(x_pad, W, a1, a2):
    BM = 2048
    return pl.pallas_call(
        _node_body,
        grid=(NP // BM,),
        in_specs=[pl.BlockSpec((BM, D), lambda i: (i, 0)),
                  pl.BlockSpec((D, D), lambda i: (0, 0)),
                  pl.BlockSpec((1, 2 * D), lambda i: (0, 0)),
                  pl.BlockSpec((1, 2 * D), lambda i: (0, 0))],
        out_specs=[pl.BlockSpec((BM, D), lambda i: (i, 0)),
                   pl.BlockSpec((BM,), lambda i: (i,)),
                   pl.BlockSpec((BM,), lambda i: (i,))],
        out_shape=[jax.ShapeDtypeStruct((NP, D), F32),
                   jax.ShapeDtypeStruct((NP,), F32),
                   jax.ShapeDtypeStruct((NP,), F32)],
    )(x_pad, W, a1, a2)


EBM = 16000


def _edge_body(p_ref, nh_ref, w_ref, a1_ref, a2_ref, v1_ref, u2_ref):
    i = pl.program_id(0)
    W = w_ref[...]
    c1 = jnp.sum(W * a1_ref[0, D:][None, :], axis=1)
    c2 = jnp.sum(W * a2_ref[0, :D][None, :], axis=1)
    v1_ref[pl.ds(i * EBM, EBM)] = jnp.sum(p_ref[...] * c1[None, :], axis=1)
    u2_ref[pl.ds(i * EBM, EBM)] = jnp.sum(nh_ref[...] * c2[None, :], axis=1)


def _edge_pre(p_h, new_h, W, a1, a2):
    BM = EBM
    return pl.pallas_call(
        _edge_body,
        grid=(E // BM,),
        in_specs=[pl.BlockSpec((BM, D), lambda i: (i, 0)),
                  pl.BlockSpec((BM, D), lambda i: (i, 0)),
                  pl.BlockSpec((D, D), lambda i: (0, 0)),
                  pl.BlockSpec((1, 2 * D), lambda i: (0, 0)),
                  pl.BlockSpec((1, 2 * D), lambda i: (0, 0))],
        out_specs=[pl.BlockSpec((E,), lambda i: (0,)),
                   pl.BlockSpec((E,), lambda i: (0,))],
        out_shape=[jax.ShapeDtypeStruct((E,), F32),
                   jax.ShapeDtypeStruct((E,), F32)],
    )(p_h, new_h, W, a1, a2)


def _combine_body(p_ref, o_ref):
    hp = p_ref[...]
    o_ref[...] = jnp.where(hp > 0, hp, jnp.exp(jnp.minimum(hp, 0.0)) - 1.0)


def _combine(p):
    BM = 2000
    return pl.pallas_call(
        _combine_body,
        grid=(N // BM,),
        in_specs=[pl.BlockSpec((BM, D), lambda i: (i, 0))],
        out_specs=pl.BlockSpec((BM, D), lambda i: (i, 0)),
        out_shape=jax.ShapeDtypeStruct((N, D), F32),
    )(p)


def _worker():
    cid = lax.axis_index("c")
    sid = lax.axis_index("s")
    return cid, sid, cid * 16 + sid


def _leaky_exp(s):
    return jnp.exp(-jnp.where(s >= 0, s, ALPHA * s))


_GDN = lax.GatherDimensionNumbers(
    offset_dims=(), collapsed_slice_dims=(0,), start_index_map=(0,))


def _splat(vec, j):
    idx = jnp.full((16, 1), j, I32)
    return lax.gather(vec, idx, _GDN, slice_sizes=(1,),
                      mode=lax.GatherScatterMode.PROMISE_IN_BOUNDS)


def _sc_attn_col(u1, v1_2d, ec0_2d, row_2d):
    @functools.partial(
        pl.kernel,
        out_type=[jax.ShapeDtypeStruct((NW, NCH, CH), F32),
                  jax.ShapeDtypeStruct((2 * NP,), F32)],
        mesh=_MESH(),
        compiler_params=pltpu.CompilerParams(needs_layout_passes=False),
        scratch_types=[
            pltpu.VMEM((NP,), F32),
            pltpu.VMEM((NCH, CH), F32),
            pltpu.VMEM((NCH, CH), I32),
            pltpu.VMEM((NCH, CH), I32),
            pltpu.VMEM((NCH, CH), F32),
            pltpu.VMEM((SEG,), F32),
            pltpu.VMEM_SHARED((NP,), F32),
            pltpu.SemaphoreType.DMA,
        ],
    )
    def kern(u1_hbm, v1_hbm, ec0_hbm, row_hbm, ex1_hbm, part_hbm,
             u1_v, v1_v, ec0_v, row_v, ex1_v, zb_v, seg_sh, sem):
        cid, sid, wid = _worker()
        pltpu.sync_copy(u1_hbm, u1_v)
        pltpu.sync_copy(v1_hbm.at[wid], v1_v)
        pltpu.sync_copy(ec0_hbm.at[wid], ec0_v)
        pltpu.sync_copy(row_hbm.at[wid], row_v)

        def zfill(i, _):
            zb_v[pl.ds(i * 16, 16)] = jnp.zeros((16,), F32)
            return 0
        lax.fori_loop(0, SEG // 16, zfill, 0)
        pltpu.sync_copy(zb_v, seg_sh.at[pl.ds(sid * SEG, SEG)])
        plsc.subcore_barrier()

        def row_fn(r, _):
            for k in range(CH // 16):
                sl = pl.ds(k * 16, 16)
                g = plsc.load_gather(u1_v, [ec0_v[r, sl]])
                ec = _leaky_exp(g + v1_v[r, sl])
                ex1_v[r, sl] = jnp.exp(ec)
            return 0
        lax.fori_loop(0, NCH, row_fn, 0)
        pltpu.sync_copy(ex1_v, ex1_hbm.at[wid])

        def scat(w, _):
            cps = []
            for j in range(5):
                ch = w * 5 + j
                cps.append(pltpu.async_copy(
                    ex1_v.at[ch], seg_sh.at[row_v.at[ch]], sem, add=True))
            for cp in cps:
                cp.wait()
            return 0
        lax.fori_loop(0, NCH // 5, scat, 0)
        plsc.subcore_barrier()
        pltpu.sync_copy(seg_sh.at[pl.ds(sid * SEG, SEG)],
                        part_hbm.at[pl.ds(cid * NP + sid * SEG, SEG)])

    return kern(u1, v1_2d, ec0_2d, row_2d)


def _sc_softmax_div(seg_part, ex1_2d, row_2d):
    @functools.partial(
        pl.kernel,
        out_type=jax.ShapeDtypeStruct((NW, NCH, CH), F32),
        mesh=_MESH(),
        compiler_params=pltpu.CompilerParams(needs_layout_passes=False),
        scratch_types=[
            pltpu.VMEM((NP,), F32),
            pltpu.VMEM((NP,), F32),
            pltpu.VMEM((NCH, CH), F32),
            pltpu.VMEM((NCH, CH), I32),
            pltpu.VMEM((NCH, CH), F32),
        ],
    )
    def kern(part_hbm, ex1_hbm, row_hbm, z_hbm,
             pa_v, pb_v, ex1_v, row_v, z_v):
        cid, sid, wid = _worker()
        pltpu.sync_copy(part_hbm.at[pl.ds(0, NP)], pa_v)
        pltpu.sync_copy(part_hbm.at[pl.ds(NP, NP)], pb_v)
        pltpu.sync_copy(ex1_hbm.at[wid], ex1_v)
        pltpu.sync_copy(row_hbm.at[wid], row_v)

        def red(i, _):
            sl = pl.ds(i * 16, 16)
            pa_v[sl] = pa_v[sl] + pb_v[sl] + 1e-16
            return 0
        lax.fori_loop(0, NP // 16, red, 0)

        def row_fn(r, _):
            for k in range(CH // 16):
                sl = pl.ds(k * 16, 16)
                ss = plsc.load_gather(pa_v, [row_v[r, sl]])
                z_v[r, sl] = ex1_v[r, sl] / ss
            return 0
        lax.fori_loop(0, NCH, row_fn, 0)
        pltpu.sync_copy(z_v, z_hbm.at[wid])

    return kern(seg_part, ex1_2d, row_2d)


def _sc_edge_row(w2n, u2_flat, z_flat, rr_2d, e1_2d, e0_2d):
    @functools.partial(
        pl.kernel,
        out_type=[jax.ShapeDtypeStruct((NW, NCH, CH), F32),
                  jax.ShapeDtypeStruct((2 * NP,), F32)],
        mesh=_MESH(),
        compiler_params=pltpu.CompilerParams(needs_layout_passes=False),
        scratch_types=[
            pltpu.VMEM((NP,), F32),
            pltpu.VMEM((NCH, CH), I32),
            pltpu.VMEM((NCH, CH), I32),
            pltpu.VMEM((NCH, CH), I32),
            pltpu.VMEM((NCH, CH), F32),
            pltpu.VMEM((NCH, CH), F32),
            pltpu.VMEM((NCH, CH), F32),
            pltpu.VMEM((SEG,), F32),
            pltpu.VMEM_SHARED((NP,), F32),
            pltpu.SemaphoreType.DMA,
        ],
    )
    def kern(w2_hbm, u2_hbm, z_hbm, rr_hbm, e1_hbm, e0_hbm, ee_hbm, part_hbm,
             w2_v, rr_v, e1_v, e0_v, u2r_v, zr_v, ee_v, zb_v, rs_sh, sem):
        cid, sid, wid = _worker()
        pltpu.sync_copy(w2_hbm, w2_v)
        pltpu.sync_copy(rr_hbm.at[wid], rr_v)
        pltpu.sync_copy(e1_hbm.at[wid], e1_v)
        pltpu.sync_copy(e0_hbm.at[wid], e0_v)

        def zfill(i, _):
            zb_v[pl.ds(i * 16, 16)] = jnp.zeros((16,), F32)
            return 0
        lax.fori_loop(0, SEG // 16, zfill, 0)
        pltpu.sync_copy(zb_v, rs_sh.at[pl.ds(sid * SEG, SEG)])
        plsc.subcore_barrier()

        def gat(w, _):
            cps = []
            for j in range(5):
                ch = w * 5 + j
                cps.append(pltpu.async_copy(
                    u2_hbm.at[rr_v.at[ch]], u2r_v.at[ch], sem))
                cps.append(pltpu.async_copy(
                    z_hbm.at[rr_v.at[ch]], zr_v.at[ch], sem))
            for cp in cps:
                cp.wait()
            return 0
        lax.fori_loop(0, NCH // 5, gat, 0)

        def row_fn(r, _):
            for k in range(CH // 16):
                sl = pl.ds(k * 16, 16)
                g = plsc.load_gather(w2_v, [e1_v[r, sl]])
                er = _leaky_exp(u2r_v[r, sl] + g)
                ee_v[r, sl] = er * zr_v[r, sl]
            return 0
        lax.fori_loop(0, NCH, row_fn, 0)
        pltpu.sync_copy(ee_v, ee_hbm.at[wid])

        def scat(w, _):
            cps = []
            for j in range(5):
                ch = w * 5 + j
                cps.append(pltpu.async_copy(
                    ee_v.at[ch], rs_sh.at[e0_v.at[ch]], sem, add=True))
            for cp in cps:
                cp.wait()
            return 0
        lax.fori_loop(0, NCH // 5, scat, 0)
        plsc.subcore_barrier()
        pltpu.sync_copy(rs_sh.at[pl.ds(sid * SEG, SEG)],
                        part_hbm.at[pl.ds(cid * NP + sid * SEG, SEG)])

    return kern(w2n, u2_flat, z_flat, rr_2d, e1_2d, e0_2d)


def _sc_aggregate(h_pad, ee_g, rs_part, e0_g, e1_g):
    HALF = NP // 2
    GS = 25
    NG = (E // 16) // (GS * CH)
    SEGH = HALF // 16

    @functools.partial(
        pl.kernel,
        out_type=[jax.ShapeDtypeStruct((16 * NG, GS, CH), F32),
                  jax.ShapeDtypeStruct((2, HALF, D), F32)],
        mesh=_MESH(),
        compiler_params=pltpu.CompilerParams(needs_layout_passes=False),
        scratch_types=[
            pltpu.VMEM((NP,), F32),
            pltpu.VMEM((SEG,), F32),
            pltpu.VMEM((GS, CH), I32),
            pltpu.VMEM((GS, CH), I32),
            pltpu.VMEM((GS, CH), F32),
            pltpu.VMEM((CH, D), F32),
            pltpu.VMEM((CH, D), F32),
            pltpu.VMEM((CH, D), F32),
            pltpu.VMEM_SHARED((HALF, D), F32),
            pltpu.SemaphoreType.DMA,
            pltpu.SemaphoreType.DMA,
        ],
    )
    def kern(h_hbm, ee_hbm, part_hbm, e0_hbm, e1_hbm, att_hbm, hp_hbm,
             rs_v, tmp_v, e0_v, e1_v, att_v, ga_v, gb_v, sb_v, hp_sh,
             sg, ss):
        cid, sid, wid = _worker()
        pltpu.sync_copy(part_hbm.at[pl.ds(0, NP)], rs_v)

        def red(k, _):
            pltpu.sync_copy(part_hbm.at[pl.ds(NP + k * SEG, SEG)], tmp_v)

            def red2(j, _):
                sl = pl.ds(j * 16, 16)
                gl = pl.ds(k * SEG + j * 16, 16)
                t = rs_v[gl] + tmp_v[sl]
                rs_v[gl] = jnp.where(t == 0, 1.0, t)
                return 0
            lax.fori_loop(0, SEG // 16, red2, 0)
            return 0
        lax.fori_loop(0, 16, red, 0)

        for i in range(16):
            for k in range(D // 16):
                ga_v[i, pl.ds(k * 16, 16)] = jnp.zeros((16,), F32)

        def zrow(j, _):
            pltpu.sync_copy(ga_v.at[pl.ds(0, 16)],
                            hp_sh.at[pl.ds(sid * SEGH + j * 16, 16)])
            return 0
        lax.fori_loop(0, SEGH // 16, zrow, 0)
        plsc.subcore_barrier()

        base = cid * HALF

        def group_fn(g, _):
            gid = sid * NG + g
            pltpu.sync_copy(ee_hbm.at[gid], att_v)
            pltpu.sync_copy(e0_hbm.at[gid], e0_v)
            pltpu.sync_copy(e1_hbm.at[gid], e1_v)

            def att_fn(r, _):
                for k in range(CH // 16):
                    sl = pl.ds(k * 16, 16)
                    rs = plsc.load_gather(rs_v, [e0_v[r, sl]])
                    att_v[r, sl] = att_v[r, sl] / rs
                return 0
            lax.fori_loop(0, GS, att_fn, 0)

            @pl.when(cid == 0)
            def _():
                pltpu.sync_copy(att_v, att_hbm.at[gid])

            def mask_fn(r, _):
                for k in range(CH // 16):
                    sl = pl.ds(k * 16, 16)
                    idx = e0_v[r, sl] - base
                    inb = (idx >= 0) & (idx < HALF)
                    e0_v[r, sl] = jnp.where(inb, idx, 0)
                    att_v[r, sl] = jnp.where(inb, att_v[r, sl], 0.0)
                return 0
            lax.fori_loop(0, GS, mask_fn, 0)

            def scale(ch, src, dst):
                def grp(g, _):
                    av = att_v[ch, pl.ds(g * 16, 16)]
                    for j in range(16):
                        spl = _splat(av, j)
                        r = g * 16 + j
                        for k in range(D // 16):
                            sl = pl.ds(k * 16, 16)
                            dst[r, sl] = src[r, sl] * spl
                    return 0
                lax.fori_loop(0, CH // 16, grp, 0)

            def drain_gather(dst):
                pltpu.make_async_copy(h_hbm.at[pl.ds(0, CH)], dst, sg).wait()

            def drain_scatter():
                pltpu.make_async_copy(
                    sb_v, hp_sh.at[pl.ds(0, CH)], ss).wait()

            pltpu.async_copy(h_hbm.at[e1_v.at[0]], ga_v, sg)

            def chunk_pair(p2, _):
                for par in range(2):
                    ch = p2 * 2 + par
                    gb = ga_v if par == 0 else gb_v
                    nxt = gb_v if par == 0 else ga_v
                    pltpu.async_copy(h_hbm.at[e1_v.at[ch + 1]], nxt, sg)
                    drain_gather(gb)

                    @pl.when(ch >= 1)
                    def _():
                        drain_scatter()
                    scale(ch, gb, sb_v)
                    pltpu.async_copy(sb_v, hp_sh.at[e0_v.at[ch]], ss,
                                     add=True)
                return 0
            lax.fori_loop(0, GS // 2, chunk_pair, 0)

            drain_gather(ga_v)
            drain_scatter()
            scale(GS - 1, ga_v, sb_v)
            pltpu.async_copy(sb_v, hp_sh.at[e0_v.at[GS - 1]], ss, add=True)
            drain_scatter()
            return 0
        lax.fori_loop(0, NG, group_fn, 0)
        plsc.subcore_barrier()
        pltpu.sync_copy(hp_sh.at[pl.ds(sid * SEGH, SEGH)],
                        hp_hbm.at[cid, pl.ds(sid * SEGH, SEGH)])

    return kern(h_pad, ee_g, rs_part, e0_g, e1_g)


def kernel(input, adj, edge, p_h, edge_col, row_i, row_resort, new_h, W, a1, a2):
    x_pad = jnp.pad(input, ((0, NP - N), (0, 0)))
    h_pad, u1, w2n = _node_pre(x_pad, W, a1, a2)
    v1, u2 = _edge_pre(p_h, new_h, W, a1, a2)

    ec0_2d = edge_col[0].reshape(NW, NCH, CH)
    row_2d = row_i.reshape(NW, NCH, CH)
    rr_2d = row_resort.reshape(NW, NCH, CH)
    e0_2d = edge[0].reshape(NW, NCH, CH)
    e1_2d = edge[1].reshape(NW, NCH, CH)

    ex1_2d, seg_part = _sc_attn_col(u1, v1.reshape(NW, NCH, CH), ec0_2d, row_2d)
    z_2d = _sc_softmax_div(seg_part, ex1_2d, row_2d)
    ee_2d, rs_part = _sc_edge_row(w2n, u2, z_2d.reshape(E), rr_2d, e1_2d, e0_2d)
    gshape = (160, 25, CH)
    att_h, hp_halves = _sc_aggregate(h_pad, ee_2d.reshape(gshape), rs_part,
                                     edge[0].reshape(gshape),
                                     edge[1].reshape(gshape))

    h_prime = _combine(hp_halves.reshape(NP, D)[:N])
    return h_prime, edge, att_h.reshape(E, 1)


# speedup vs baseline: 10.0365x; 1.0002x over previous
import functools

import jax
import jax.numpy as jnp
from jax import lax
from jax.experimental import pallas as pl
from jax.experimental.pallas import tpu as pltpu
from jax.experimental.pallas import tpu_sc as plsc

N = 10000
NP = 10240
E = 320000
D = 128
ALPHA = 0.2

NW = 32
C = E // NW
CH = 80
NCH = C // CH
ER = E // CH
SEG = NP // 16
F32 = jnp.float32
I32 = jnp.int32

_mesh_cache = []


def _MESH():
    if not _mesh_cache:
        _mesh_cache.append(plsc.VectorSubcoreMesh(
            core_axis_name="c", subcore_axis_name="s"))
    return _mesh_cache[0]


def _node_body(x_ref, w_ref, a1_ref, a2_ref, h_ref, u1_ref, w2_ref):
    h = jnp.dot(x_ref[...], w_ref[...], preferred_element_type=F32)
    h_ref[...] = h
    u1_ref[...] = jnp.sum(h * a1_ref[0, :D][None, :], axis=1)
    w2_ref[...] = jnp.sum(h * a2_ref[0, D:][None, :], axis=1)


def _node_pre(x_pad, W, a1, a2):
    BM = 2048
    return pl.pallas_call(
        _node_body,
        grid=(NP // BM,),
        in_specs=[pl.BlockSpec((BM, D), lambda i: (i, 0)),
                  pl.BlockSpec((D, D), lambda i: (0, 0)),
                  pl.BlockSpec((1, 2 * D), lambda i: (0, 0)),
                  pl.BlockSpec((1, 2 * D), lambda i: (0, 0))],
        out_specs=[pl.BlockSpec((BM, D), lambda i: (i, 0)),
                   pl.BlockSpec((BM,), lambda i: (i,)),
                   pl.BlockSpec((BM,), lambda i: (i,))],
        out_shape=[jax.ShapeDtypeStruct((NP, D), F32),
                   jax.ShapeDtypeStruct((NP,), F32),
                   jax.ShapeDtypeStruct((NP,), F32)],
    )(x_pad, W, a1, a2)


EBM = 16000


def _edge_body(p_ref, nh_ref, w_ref, a1_ref, a2_ref, v1_ref, u2_ref):
    i = pl.program_id(0)
    W = w_ref[...]
    c1 = jnp.sum(W * a1_ref[0, D:][None, :], axis=1)
    c2 = jnp.sum(W * a2_ref[0, :D][None, :], axis=1)
    v1_ref[pl.ds(i * EBM, EBM)] = jnp.dot(p_ref[...], c1,
                                          preferred_element_type=F32)
    u2_ref[pl.ds(i * EBM, EBM)] = jnp.dot(nh_ref[...], c2,
                                          preferred_element_type=F32)


def _edge_pre(p_h, new_h, W, a1, a2):
    BM = EBM
    return pl.pallas_call(
        _edge_body,
        grid=(E // BM,),
        in_specs=[pl.BlockSpec((BM, D), lambda i: (i, 0)),
                  pl.BlockSpec((BM, D), lambda i: (i, 0)),
                  pl.BlockSpec((D, D), lambda i: (0, 0)),
                  pl.BlockSpec((1, 2 * D), lambda i: (0, 0)),
                  pl.BlockSpec((1, 2 * D), lambda i: (0, 0))],
        out_specs=[pl.BlockSpec((E,), lambda i: (0,)),
                   pl.BlockSpec((E,), lambda i: (0,))],
        out_shape=[jax.ShapeDtypeStruct((E,), F32),
                   jax.ShapeDtypeStruct((E,), F32)],
    )(p_h, new_h, W, a1, a2)


def _combine_body(p_ref, o_ref):
    hp = p_ref[...]
    o_ref[...] = jnp.where(hp > 0, hp, jnp.exp(jnp.minimum(hp, 0.0)) - 1.0)


def _combine(p):
    BM = 2000
    return pl.pallas_call(
        _combine_body,
        grid=(N // BM,),
        in_specs=[pl.BlockSpec((BM, D), lambda i: (i, 0))],
        out_specs=pl.BlockSpec((BM, D), lambda i: (i, 0)),
        out_shape=jax.ShapeDtypeStruct((N, D), F32),
    )(p)


def _worker():
    cid = lax.axis_index("c")
    sid = lax.axis_index("s")
    return cid, sid, cid * 16 + sid


def _leaky_exp(s):
    return jnp.exp(-jnp.where(s >= 0, s, ALPHA * s))


_GDN = lax.GatherDimensionNumbers(
    offset_dims=(), collapsed_slice_dims=(0,), start_index_map=(0,))


def _splat(vec, j):
    idx = jnp.full((16, 1), j, I32)
    return lax.gather(vec, idx, _GDN, slice_sizes=(1,),
                      mode=lax.GatherScatterMode.PROMISE_IN_BOUNDS)


def _sc_attn_col(u1, v1_2d, ec0_2d, row_2d):
    @functools.partial(
        pl.kernel,
        out_type=[jax.ShapeDtypeStruct((NW, NCH, CH), F32),
                  jax.ShapeDtypeStruct((2 * NP,), F32)],
        mesh=_MESH(),
        compiler_params=pltpu.CompilerParams(needs_layout_passes=False),
        scratch_types=[
            pltpu.VMEM((NP,), F32),
            pltpu.VMEM((NCH, CH), F32),
            pltpu.VMEM((NCH, CH), I32),
            pltpu.VMEM((NCH, CH), I32),
            pltpu.VMEM((NCH, CH), F32),
            pltpu.VMEM((SEG,), F32),
            pltpu.VMEM_SHARED((NP,), F32),
            pltpu.SemaphoreType.DMA,
        ],
    )
    def kern(u1_hbm, v1_hbm, ec0_hbm, row_hbm, ex1_hbm, part_hbm,
             u1_v, v1_v, ec0_v, row_v, ex1_v, zb_v, seg_sh, sem):
        cid, sid, wid = _worker()
        pltpu.sync_copy(u1_hbm, u1_v)
        pltpu.sync_copy(v1_hbm.at[wid], v1_v)
        pltpu.sync_copy(ec0_hbm.at[wid], ec0_v)
        pltpu.sync_copy(row_hbm.at[wid], row_v)

        def zfill(i, _):
            zb_v[pl.ds(i * 16, 16)] = jnp.zeros((16,), F32)
            return 0
        lax.fori_loop(0, SEG // 16, zfill, 0)
        pltpu.sync_copy(zb_v, seg_sh.at[pl.ds(sid * SEG, SEG)])
        plsc.subcore_barrier()

        def row_fn(r, _):
            for k in range(CH // 16):
                sl = pl.ds(k * 16, 16)
                g = plsc.load_gather(u1_v, [ec0_v[r, sl]])
                ec = _leaky_exp(g + v1_v[r, sl])
                ex1_v[r, sl] = jnp.exp(ec)
            return 0
        lax.fori_loop(0, NCH, row_fn, 0)
        pltpu.sync_copy(ex1_v, ex1_hbm.at[wid])

        def scat(w, _):
            cps = []
            for j in range(5):
                ch = w * 5 + j
                cps.append(pltpu.async_copy(
                    ex1_v.at[ch], seg_sh.at[row_v.at[ch]], sem, add=True))
            for cp in cps:
                cp.wait()
            return 0
        lax.fori_loop(0, NCH // 5, scat, 0)
        plsc.subcore_barrier()
        pltpu.sync_copy(seg_sh.at[pl.ds(sid * SEG, SEG)],
                        part_hbm.at[pl.ds(cid * NP + sid * SEG, SEG)])

    return kern(u1, v1_2d, ec0_2d, row_2d)


def _sc_softmax_div(seg_part, ex1_2d, row_2d):
    @functools.partial(
        pl.kernel,
        out_type=jax.ShapeDtypeStruct((NW, NCH, CH), F32),
        mesh=_MESH(),
        compiler_params=pltpu.CompilerParams(needs_layout_passes=False),
        scratch_types=[
            pltpu.VMEM((NP,), F32),
            pltpu.VMEM((NP,), F32),
            pltpu.VMEM((NCH, CH), F32),
            pltpu.VMEM((NCH, CH), I32),
            pltpu.VMEM((NCH, CH), F32),
        ],
    )
    def kern(part_hbm, ex1_hbm, row_hbm, z_hbm,
             pa_v, pb_v, ex1_v, row_v, z_v):
        cid, sid, wid = _worker()
        pltpu.sync_copy(part_hbm.at[pl.ds(0, NP)], pa_v)
        pltpu.sync_copy(part_hbm.at[pl.ds(NP, NP)], pb_v)
        pltpu.sync_copy(ex1_hbm.at[wid], ex1_v)
        pltpu.sync_copy(row_hbm.at[wid], row_v)

        def red(i, _):
            sl = pl.ds(i * 16, 16)
            pa_v[sl] = pa_v[sl] + pb_v[sl] + 1e-16
            return 0
        lax.fori_loop(0, NP // 16, red, 0)

        def row_fn(r, _):
            for k in range(CH // 16):
                sl = pl.ds(k * 16, 16)
                ss = plsc.load_gather(pa_v, [row_v[r, sl]])
                z_v[r, sl] = ex1_v[r, sl] / ss
            return 0
        lax.fori_loop(0, NCH, row_fn, 0)
        pltpu.sync_copy(z_v, z_hbm.at[wid])

    return kern(seg_part, ex1_2d, row_2d)


def _sc_edge_row(w2n, u2_flat, z_flat, rr_2d, e1_2d, e0_2d):
    @functools.partial(
        pl.kernel,
        out_type=[jax.ShapeDtypeStruct((NW, NCH, CH), F32),
                  jax.ShapeDtypeStruct((2 * NP,), F32)],
        mesh=_MESH(),
        compiler_params=pltpu.CompilerParams(needs_layout_passes=False),
        scratch_types=[
            pltpu.VMEM((NP,), F32),
            pltpu.VMEM((NCH, CH), I32),
            pltpu.VMEM((NCH, CH), I32),
            pltpu.VMEM((NCH, CH), I32),
            pltpu.VMEM((NCH, CH), F32),
            pltpu.VMEM((NCH, CH), F32),
            pltpu.VMEM((NCH, CH), F32),
            pltpu.VMEM((SEG,), F32),
            pltpu.VMEM_SHARED((NP,), F32),
            pltpu.SemaphoreType.DMA,
        ],
    )
    def kern(w2_hbm, u2_hbm, z_hbm, rr_hbm, e1_hbm, e0_hbm, ee_hbm, part_hbm,
             w2_v, rr_v, e1_v, e0_v, u2r_v, zr_v, ee_v, zb_v, rs_sh, sem):
        cid, sid, wid = _worker()
        pltpu.sync_copy(w2_hbm, w2_v)
        pltpu.sync_copy(rr_hbm.at[wid], rr_v)
        pltpu.sync_copy(e1_hbm.at[wid], e1_v)
        pltpu.sync_copy(e0_hbm.at[wid], e0_v)

        def zfill(i, _):
            zb_v[pl.ds(i * 16, 16)] = jnp.zeros((16,), F32)
            return 0
        lax.fori_loop(0, SEG // 16, zfill, 0)
        pltpu.sync_copy(zb_v, rs_sh.at[pl.ds(sid * SEG, SEG)])
        plsc.subcore_barrier()

        def gat(w, _):
            cps = []
            for j in range(5):
                ch = w * 5 + j
                cps.append(pltpu.async_copy(
                    u2_hbm.at[rr_v.at[ch]], u2r_v.at[ch], sem))
                cps.append(pltpu.async_copy(
                    z_hbm.at[rr_v.at[ch]], zr_v.at[ch], sem))
            for cp in cps:
                cp.wait()
            return 0
        lax.fori_loop(0, NCH // 5, gat, 0)

        def row_fn(r, _):
            for k in range(CH // 16):
                sl = pl.ds(k * 16, 16)
                g = plsc.load_gather(w2_v, [e1_v[r, sl]])
                er = _leaky_exp(u2r_v[r, sl] + g)
                ee_v[r, sl] = er * zr_v[r, sl]
            return 0
        lax.fori_loop(0, NCH, row_fn, 0)
        pltpu.sync_copy(ee_v, ee_hbm.at[wid])

        def scat(w, _):
            cps = []
            for j in range(5):
                ch = w * 5 + j
                cps.append(pltpu.async_copy(
                    ee_v.at[ch], rs_sh.at[e0_v.at[ch]], sem, add=True))
            for cp in cps:
                cp.wait()
            return 0
        lax.fori_loop(0, NCH // 5, scat, 0)
        plsc.subcore_barrier()
        pltpu.sync_copy(rs_sh.at[pl.ds(sid * SEG, SEG)],
                        part_hbm.at[pl.ds(cid * NP + sid * SEG, SEG)])

    return kern(w2n, u2_flat, z_flat, rr_2d, e1_2d, e0_2d)


def _sc_aggregate(h_pad, ee_g, rs_part, e0_g, e1_g):
    HALF = NP // 2
    GS = 25
    NG = (E // 16) // (GS * CH)
    SEGH = HALF // 16

    @functools.partial(
        pl.kernel,
        out_type=[jax.ShapeDtypeStruct((16 * NG, GS, CH), F32),
                  jax.ShapeDtypeStruct((2, HALF, D), F32)],
        mesh=_MESH(),
        compiler_params=pltpu.CompilerParams(needs_layout_passes=False),
        scratch_types=[
            pltpu.VMEM((NP,), F32),
            pltpu.VMEM((SEG,), F32),
            pltpu.VMEM((GS, CH), I32),
            pltpu.VMEM((GS, CH), I32),
            pltpu.VMEM((GS, CH), F32),
            pltpu.VMEM((CH, D), F32),
            pltpu.VMEM((CH, D), F32),
            pltpu.VMEM((CH, D), F32),
            pltpu.VMEM_SHARED((HALF, D), F32),
            pltpu.SemaphoreType.DMA,
            pltpu.SemaphoreType.DMA,
        ],
    )
    def kern(h_hbm, ee_hbm, part_hbm, e0_hbm, e1_hbm, att_hbm, hp_hbm,
             rs_v, tmp_v, e0_v, e1_v, att_v, ga_v, gb_v, sb_v, hp_sh,
             sg, ss):
        cid, sid, wid = _worker()
        pltpu.sync_copy(part_hbm.at[pl.ds(0, NP)], rs_v)

        def red(k, _):
            pltpu.sync_copy(part_hbm.at[pl.ds(NP + k * SEG, SEG)], tmp_v)

            def red2(j, _):
                sl = pl.ds(j * 16, 16)
                gl = pl.ds(k * SEG + j * 16, 16)
                t = rs_v[gl] + tmp_v[sl]
                rs_v[gl] = jnp.where(t == 0, 1.0, t)
                return 0
            lax.fori_loop(0, SEG // 16, red2, 0)
            return 0
        lax.fori_loop(0, 16, red, 0)

        for i in range(16):
            for k in range(D // 16):
                ga_v[i, pl.ds(k * 16, 16)] = jnp.zeros((16,), F32)

        def zrow(j, _):
            pltpu.sync_copy(ga_v.at[pl.ds(0, 16)],
                            hp_sh.at[pl.ds(sid * SEGH + j * 16, 16)])
            return 0
        lax.fori_loop(0, SEGH // 16, zrow, 0)
        plsc.subcore_barrier()

        base = cid * HALF

        def group_fn(g, _):
            gid = sid * NG + g
            pltpu.sync_copy(ee_hbm.at[gid], att_v)
            pltpu.sync_copy(e0_hbm.at[gid], e0_v)
            pltpu.sync_copy(e1_hbm.at[gid], e1_v)

            def att_fn(r, _):
                for k in range(CH // 16):
                    sl = pl.ds(k * 16, 16)
                    rs = plsc.load_gather(rs_v, [e0_v[r, sl]])
                    att_v[r, sl] = att_v[r, sl] / rs
                return 0
            lax.fori_loop(0, GS, att_fn, 0)

            @pl.when(cid == 0)
            def _():
                pltpu.sync_copy(att_v, att_hbm.at[gid])

            def mask_fn(r, _):
                for k in range(CH // 16):
                    sl = pl.ds(k * 16, 16)
                    idx = e0_v[r, sl] - base
                    inb = (idx >= 0) & (idx < HALF)
                    e0_v[r, sl] = jnp.where(inb, idx, 0)
                    att_v[r, sl] = jnp.where(inb, att_v[r, sl], 0.0)
                return 0
            lax.fori_loop(0, GS, mask_fn, 0)

            def scale(ch, src, dst):
                def grp(g, _):
                    av = att_v[ch, pl.ds(g * 16, 16)]
                    for j in range(16):
                        spl = _splat(av, j)
                        r = g * 16 + j
                        for k in range(D // 16):
                            sl = pl.ds(k * 16, 16)
                            dst[r, sl] = src[r, sl] * spl
                    return 0
                lax.fori_loop(0, CH // 16, grp, 0)

            def drain_gather(dst):
                pltpu.make_async_copy(h_hbm.at[pl.ds(0, CH)], dst, sg).wait()

            def drain_scatter():
                pltpu.make_async_copy(
                    sb_v, hp_sh.at[pl.ds(0, CH)], ss).wait()

            pltpu.async_copy(h_hbm.at[e1_v.at[0]], ga_v, sg)

            def chunk_pair(p2, _):
                for par in range(2):
                    ch = p2 * 2 + par
                    gb = ga_v if par == 0 else gb_v
                    nxt = gb_v if par == 0 else ga_v
                    pltpu.async_copy(h_hbm.at[e1_v.at[ch + 1]], nxt, sg)
                    drain_gather(gb)

                    @pl.when(ch >= 1)
                    def _():
                        drain_scatter()
                    scale(ch, gb, sb_v)
                    pltpu.async_copy(sb_v, hp_sh.at[e0_v.at[ch]], ss,
                                     add=True)
                return 0
            lax.fori_loop(0, GS // 2, chunk_pair, 0)

            drain_gather(ga_v)
            drain_scatter()
            scale(GS - 1, ga_v, sb_v)
            pltpu.async_copy(sb_v, hp_sh.at[e0_v.at[GS - 1]], ss, add=True)
            drain_scatter()
            return 0
        lax.fori_loop(0, NG, group_fn, 0)
        plsc.subcore_barrier()
        pltpu.sync_copy(hp_sh.at[pl.ds(sid * SEGH, SEGH)],
                        hp_hbm.at[cid, pl.ds(sid * SEGH, SEGH)])

    return kern(h_pad, ee_g, rs_part, e0_g, e1_g)


def kernel(input, adj, edge, p_h, edge_col, row_i, row_resort, new_h, W, a1, a2):
    x_pad = jnp.pad(input, ((0, NP - N), (0, 0)))
    h_pad, u1, w2n = _node_pre(x_pad, W, a1, a2)
    v1, u2 = _edge_pre(p_h, new_h, W, a1, a2)

    ec0_2d = edge_col[0].reshape(NW, NCH, CH)
    row_2d = row_i.reshape(NW, NCH, CH)
    rr_2d = row_resort.reshape(NW, NCH, CH)
    e0_2d = edge[0].reshape(NW, NCH, CH)
    e1_2d = edge[1].reshape(NW, NCH, CH)

    ex1_2d, seg_part = _sc_attn_col(u1, v1.reshape(NW, NCH, CH), ec0_2d, row_2d)
    z_2d = _sc_softmax_div(seg_part, ex1_2d, row_2d)
    ee_2d, rs_part = _sc_edge_row(w2n, u2, z_2d.reshape(E), rr_2d, e1_2d, e0_2d)
    gshape = (160, 25, CH)
    att_h, hp_halves = _sc_aggregate(h_pad, ee_2d.reshape(gshape), rs_part,
                                     edge[0].reshape(gshape),
                                     edge[1].reshape(gshape))

    h_prime = _combine(hp_halves.reshape(NP, D)[:N])
    return h_prime, edge, att_h.reshape(E, 1)
